# baseline (speedup 1.0000x reference)
import functools

import jax
import jax.numpy as jnp
from jax.experimental import pallas as pl
from jax.experimental.pallas import tpu as pltpu

_EPS = 1e-12
_BLOCK_ROWS = 512


def _ln_block(x_ref, g_ref, b_ref, o_ref):
    y = x_ref[...] + x_ref[...]
    mean = jnp.mean(y, axis=-1, keepdims=True)
    c = y - mean
    var = jnp.mean(c * c, axis=-1, keepdims=True)
    normed = c * jax.lax.rsqrt(var + _EPS)
    o_ref[...] = normed * g_ref[...] + b_ref[...]


@functools.partial(jax.jit, static_argnames=())
def kernel(x, lengths, gamma, beta):
    del lengths
    s, b, d = x.shape
    rows = s * b
    x2 = x.reshape(rows, d)
    br = _BLOCK_ROWS if rows % _BLOCK_ROWS == 0 else rows
    out = pl.pallas_call(
        _ln_block,
        grid=(rows // br,),
        in_specs=[
            pl.BlockSpec((br, d), lambda i: (i, 0)),
            pl.BlockSpec((1, d), lambda i: (0, 0)),
            pl.BlockSpec((1, d), lambda i: (0, 0)),
        ],
        out_specs=pl.BlockSpec((br, d), lambda i: (i, 0)),
        out_shape=jax.ShapeDtypeStruct((rows, d), x.dtype),
        compiler_params=pltpu.CompilerParams(
            dimension_semantics=("parallel",),
        ),
    )(x2, gamma.reshape(1, d), beta.reshape(1, d))
    return out.reshape(s, b, d)


# speedup vs baseline: 1.0092x; 1.0092x over previous
import functools

import jax
import jax.numpy as jnp
from jax.experimental import pallas as pl
from jax.experimental.pallas import tpu as pltpu

_EPS = 1e-12
_BLOCK_ROWS = 512


def _ln_block(x_ref, g_ref, b_ref, o_ref):
    x = x_ref[...]
    inv_n = 1.0 / x.shape[-1]
    s1 = jnp.sum(x, axis=-1, keepdims=True)
    s2 = jnp.sum(x * x, axis=-1, keepdims=True)
    mean = s1 * inv_n
    var = s2 * inv_n - mean * mean
    p = jax.lax.rsqrt(var + 0.25 * _EPS)
    q = mean * p
    o_ref[...] = (x * p - q) * g_ref[...] + b_ref[...]


@functools.partial(jax.jit, static_argnames=())
def kernel(x, lengths, gamma, beta):
    del lengths
    s, b, d = x.shape
    rows = s * b
    x2 = x.reshape(rows, d)
    br = _BLOCK_ROWS if rows % _BLOCK_ROWS == 0 else rows
    out = pl.pallas_call(
        _ln_block,
        grid=(rows // br,),
        in_specs=[
            pl.BlockSpec((br, d), lambda i: (i, 0)),
            pl.BlockSpec((1, d), lambda i: (0, 0)),
            pl.BlockSpec((1, d), lambda i: (0, 0)),
        ],
        out_specs=pl.BlockSpec((br, d), lambda i: (i, 0)),
        out_shape=jax.ShapeDtypeStruct((rows, d), x.dtype),
        compiler_params=pltpu.CompilerParams(
            dimension_semantics=("parallel",),
        ),
    )(x2, gamma.reshape(1, d), beta.reshape(1, d))
    return out.reshape(s, b, d)


# speedup vs baseline: 1.0473x; 1.0378x over previous
import functools

import jax
import jax.numpy as jnp
from jax.experimental import pallas as pl
from jax.experimental.pallas import tpu as pltpu

_EPS = 1e-12
_BLOCK_ROWS = 1024


def _ln_block(x_ref, g_ref, b_ref, o_ref):
    x = x_ref[...]
    inv_n = 1.0 / x.shape[-1]
    s1 = jnp.sum(x, axis=-1, keepdims=True)
    s2 = jnp.sum(x * x, axis=-1, keepdims=True)
    mean = s1 * inv_n
    var = s2 * inv_n - mean * mean
    p = jax.lax.rsqrt(var + 0.25 * _EPS)
    q = mean * p
    o_ref[...] = (x * p - q) * g_ref[...] + b_ref[...]


@functools.partial(jax.jit, static_argnames=())
def kernel(x, lengths, gamma, beta):
    del lengths
    s, b, d = x.shape
    rows = s * b
    x2 = x.reshape(rows, d)
    br = _BLOCK_ROWS if rows % _BLOCK_ROWS == 0 else rows
    out = pl.pallas_call(
        _ln_block,
        grid=(rows // br,),
        in_specs=[
            pl.BlockSpec((br, d), lambda i: (i, 0)),
            pl.BlockSpec((1, d), lambda i: (0, 0)),
            pl.BlockSpec((1, d), lambda i: (0, 0)),
        ],
        out_specs=pl.BlockSpec((br, d), lambda i: (i, 0)),
        out_shape=jax.ShapeDtypeStruct((rows, d), x.dtype),
        compiler_params=pltpu.CompilerParams(
            dimension_semantics=("parallel",),
        ),
    )(x2, gamma.reshape(1, d), beta.reshape(1, d))
    return out.reshape(s, b, d)


# speedup vs baseline: 1.0530x; 1.0054x over previous
import functools

import jax
import jax.numpy as jnp
from jax.experimental import pallas as pl
from jax.experimental.pallas import tpu as pltpu

_EPS = 1e-12
_BLOCK_ROWS = 2048


def _ln_block(x_ref, g_ref, b_ref, o_ref):
    x = x_ref[...]
    inv_n = 1.0 / x.shape[-1]
    s1 = jnp.sum(x, axis=-1, keepdims=True)
    s2 = jnp.sum(x * x, axis=-1, keepdims=True)
    mean = s1 * inv_n
    var = s2 * inv_n - mean * mean
    p = jax.lax.rsqrt(var + 0.25 * _EPS)
    q = mean * p
    o_ref[...] = (x * p - q) * g_ref[...] + b_ref[...]


@functools.partial(jax.jit, static_argnames=())
def kernel(x, lengths, gamma, beta):
    del lengths
    s, b, d = x.shape
    rows = s * b
    x2 = x.reshape(rows, d)
    br = _BLOCK_ROWS if rows % _BLOCK_ROWS == 0 else rows
    out = pl.pallas_call(
        _ln_block,
        grid=(rows // br,),
        in_specs=[
            pl.BlockSpec((br, d), lambda i: (i, 0)),
            pl.BlockSpec((1, d), lambda i: (0, 0)),
            pl.BlockSpec((1, d), lambda i: (0, 0)),
        ],
        out_specs=pl.BlockSpec((br, d), lambda i: (i, 0)),
        out_shape=jax.ShapeDtypeStruct((rows, d), x.dtype),
        compiler_params=pltpu.CompilerParams(
            dimension_semantics=("parallel",),
        ),
    )(x2, gamma.reshape(1, d), beta.reshape(1, d))
    return out.reshape(s, b, d)
